# baseline (speedup 1.0000x reference)
import functools

import jax
import jax.numpy as jnp
from jax import lax
from jax.experimental import pallas as pl
from jax.experimental.pallas import tpu as pltpu
from jax.experimental.pallas import tpu_sc as plsc

_NUM_CORES = 2
_NUM_SUBCORES = 16
_NUM_WORKERS = _NUM_CORES * _NUM_SUBCORES
_REP = 8
_BB = 128


def kernel(sequence, table):
    batch, seq = sequence.shape
    max_len, hidden = table.shape
    row = seq * hidden
    half = batch // 2
    b_per_w = half // _NUM_WORKERS
    n_out_dma = b_per_w // _REP
    chunk = row // _NUM_WORKERS

    tab_flat = table.reshape(-1)

    @functools.partial(
        pl.kernel,
        mesh=plsc.VectorSubcoreMesh(core_axis_name="c", subcore_axis_name="s"),
        out_type=jax.ShapeDtypeStruct((row,), jnp.float32),
        scratch_types=[
            pltpu.VMEM((chunk,), jnp.float32),
            pltpu.SemaphoreType.DMA,
        ],
    )
    def sc_lookup(tab_hbm, out_hbm, vbuf, sem):
        wid = lax.axis_index("s") * _NUM_CORES + lax.axis_index("c")
        off = wid * chunk
        pltpu.async_copy(tab_hbm.at[pl.ds(off, chunk)], vbuf, sem).wait()
        pltpu.async_copy(vbuf, out_hbm.at[pl.ds(off, chunk)], sem).wait()

    @functools.partial(
        pl.kernel,
        mesh=plsc.VectorSubcoreMesh(core_axis_name="c", subcore_axis_name="s"),
        out_type=jax.ShapeDtypeStruct((half, row), jnp.float32),
        scratch_types=[
            pltpu.VMEM((_REP, row), jnp.float32),
            pltpu.SemaphoreType.DMA,
        ],
    )
    def sc_bcast(tab_hbm, out_hbm, buf, sem):
        wid = lax.axis_index("s") * _NUM_CORES + lax.axis_index("c")
        base = wid * b_per_w
        fills = [
            pltpu.async_copy(tab_hbm.at[pl.ds(0, row)], buf.at[r], sem)
            for r in range(_REP)
        ]
        for f in fills:
            f.wait()
        outs = [
            pltpu.async_copy(buf, out_hbm.at[pl.ds(base + i * _REP, _REP)], sem)
            for i in range(n_out_dma)
        ]
        for o in outs:
            o.wait()

    stage = sc_lookup(tab_flat).reshape(1, row)
    sc_half = sc_bcast(tab_flat)

    def body(s_ref, o_ref):
        o_ref[...] = jnp.broadcast_to(s_ref[...], (_BB, row))

    tc_half = pl.pallas_call(
        body,
        grid=(half // _BB,),
        in_specs=[pl.BlockSpec((1, row), lambda i: (0, 0))],
        out_specs=pl.BlockSpec((_BB, row), lambda i: (i, 0)),
        out_shape=jax.ShapeDtypeStruct((half, row), jnp.float32),
    )(stage)

    out2d = jnp.concatenate([tc_half, sc_half], axis=0)
    return out2d.reshape(batch, seq, hidden)


# speedup vs baseline: 1.4615x; 1.4615x over previous
import functools

import jax
import jax.numpy as jnp
from jax import lax
from jax.experimental import pallas as pl
from jax.experimental.pallas import tpu as pltpu
from jax.experimental.pallas import tpu_sc as plsc

_NUM_CORES = 2
_NUM_SUBCORES = 16
_NUM_WORKERS = _NUM_CORES * _NUM_SUBCORES

_BB = 256


def kernel(sequence, table):
    batch, seq = sequence.shape
    max_len, hidden = table.shape
    row = seq * hidden
    chunk = row // _NUM_WORKERS

    tab_flat = table.reshape(-1)

    @functools.partial(
        pl.kernel,
        mesh=plsc.VectorSubcoreMesh(core_axis_name="c", subcore_axis_name="s"),
        out_type=jax.ShapeDtypeStruct((row,), jnp.float32),
        scratch_types=[
            pltpu.VMEM((chunk,), jnp.float32),
            pltpu.SemaphoreType.DMA,
        ],
    )
    def sc_lookup(tab_hbm, out_hbm, vbuf, sem):
        wid = lax.axis_index("s") * _NUM_CORES + lax.axis_index("c")
        off = wid * chunk
        pltpu.async_copy(tab_hbm.at[pl.ds(off, chunk)], vbuf, sem).wait()
        pltpu.async_copy(vbuf, out_hbm.at[pl.ds(off, chunk)], sem).wait()

    stage = sc_lookup(tab_flat).reshape(1, row)

    def body(s_ref, o_ref):
        o_ref[...] = jnp.broadcast_to(s_ref[...], (_BB, row))

    out = pl.pallas_call(
        body,
        grid=(batch // _BB,),
        in_specs=[pl.BlockSpec((1, row), lambda i: (0, 0))],
        out_specs=pl.BlockSpec((_BB, row), lambda i: (i, 0)),
        out_shape=jax.ShapeDtypeStruct((batch, row), jnp.float32),
    )(stage)
    return out.reshape(batch, seq, hidden)
